# baseline (speedup 1.0000x reference)
import functools

import jax
import jax.numpy as jnp
import numpy as np
from jax.experimental import pallas as pl
from jax.experimental.pallas import tpu as pltpu

B = 4096
NINP = 512
NHID = 1024
NB_OUT = 8
TOPK = 4
BS_OUT = NHID // NB_OUT
ATT_OUT = BS_OUT * 4
NH = 4
DH = 16
TILE = 256
F32 = jnp.float32
BF16 = jnp.bfloat16

_R_NP = (np.arange(512)[:, None] // DH == np.arange(32)[None, :]).astype(np.float32)
_E8_NP = (np.arange(8)[:, None] == np.arange(1024)[None, :] // BS_OUT).astype(np.float32)


def _fused_kernel(x_ref, h_ref, c_ref, sc_ref, a1_ref,
                  wv1_ref, e8_ref, wih_ref, whh_ref, bih_ref, bhh_ref,
                  wqkv_ref, r_ref, rt_ref,
                  fgbd_ref, fgb_ref,
                  hx_out_ref, cx_out_ref, mask_out_ref):
    x = x_ref[:]
    h = h_ref[:]
    c = c_ref[:]
    sc = sc_ref[:]
    a1 = a1_ref[:]

    v1 = jnp.dot(x.astype(BF16), wv1_ref[:], preferred_element_type=F32).astype(BF16)
    a1b = a1.astype(BF16)
    cols = [a1b[:, b:b + 1] * v1 for b in range(NB_OUT)]
    inp_flat = jnp.concatenate(cols, axis=1)

    gi = jnp.dot(inp_flat, wih_ref[:], preferred_element_type=F32) + bih_ref[:]
    gh = jnp.dot(h.astype(BF16), whh_ref[:], preferred_element_type=F32) + bhh_ref[:]

    rank = jnp.zeros((TILE, NB_OUT), jnp.int32)
    bidx = jax.lax.broadcasted_iota(jnp.int32, (TILE, NB_OUT), 1)
    for j in range(NB_OUT):
        scj = sc[:, j:j + 1]
        ahead = (scj > sc) | ((scj == sc) & (j < bidx))
        rank = rank + ahead.astype(jnp.int32)
    maskblk = (rank >= (NB_OUT - TOPK)).astype(F32)
    mask = jnp.dot(maskblk.astype(BF16), e8_ref[:],
                   preferred_element_type=F32)
    r = jax.nn.sigmoid((gi[:, :NHID] + gh[:, :NHID]).astype(BF16))
    z = jax.nn.sigmoid((gi[:, NHID:2 * NHID] + gh[:, NHID:2 * NHID]).astype(BF16))
    n = jnp.tanh((gi[:, 2 * NHID:] + r * gh[:, 2 * NHID:].astype(BF16)))
    hgru = (1.0 - z.astype(F32)) * n.astype(F32) + z.astype(F32) * h

    hgb = hgru.astype(BF16)
    qkv = jnp.dot(hgb, wqkv_ref[:], preferred_element_type=F32).astype(BF16)
    qc = qkv[:, :ATT_OUT]
    kc = qkv[:, ATT_OUT:2 * ATT_OUT]
    vc = qkv[:, 2 * ATT_OUT:]
    logits = []
    for k in range(NB_OUT):
        kct = jnp.tile(kc[:, k * 64:(k + 1) * 64], (1, NB_OUT))
        prod = qc * kct
        logits.append(jnp.dot(prod, r_ref[:],
                              preferred_element_type=F32) * 0.25)
    m = logits[0]
    for k in range(1, NB_OUT):
        m = jnp.maximum(m, logits[k])
    es = [jnp.exp(lg - m) for lg in logits]
    s = es[0]
    for k in range(1, NB_OUT):
        s = s + es[k]
    sinv = 1.0 / s
    oc = jnp.zeros((TILE, ATT_OUT), BF16)
    for k in range(NB_OUT):
        wk = es[k] * sinv
        wexp = jnp.dot(wk.astype(BF16), rt_ref[:],
                       preferred_element_type=F32).astype(BF16)
        vct = jnp.tile(vc[:, k * 64:(k + 1) * 64], (1, NB_OUT))
        oc = oc + wexp * vct

    fg = jnp.dot(oc, fgbd_ref[:], preferred_element_type=F32) + fgb_ref[:]
    fc_out = fg[:, :NHID]
    gate = jax.nn.sigmoid(fg[:, NHID:])
    h_new = hgru + gate * jnp.tanh(fc_out)

    one_m = 1.0 - mask
    hx_out_ref[:] = mask * h_new + one_m * h
    cx_out_ref[:] = mask * hgru + one_m * c
    mask_out_ref[:] = mask


def _full(shape):
    nd = len(shape)
    return pl.BlockSpec(shape, lambda i: (0,) * nd)


def kernel(inp, hx, cx, step, Wq_i, Wk_i, Wv_i, Wq_c, Wk_c, Wv_c, fc_w, fc_b, gate_w, gate_b, w_ih, w_hh, b_ih, b_hh):
    del step
    bsz = inp.shape[0]
    inp_use = inp.reshape(bsz, 1, NINP)
    inp_use = jnp.concatenate([jnp.zeros_like(inp_use[:, 0:1, :]), inp_use], axis=1)
    q = jnp.einsum('bnd,nde->bne', hx.reshape(bsz, NB_OUT, BS_OUT), Wq_i)
    k = jnp.einsum('bnd,nde->bne', inp_use, Wk_i)
    iatt = jax.nn.softmax(jnp.einsum('bqd,bkd->bqk', q, k) / np.sqrt(64.0), axis=-1)
    score = iatt[:, :, 0]
    a1 = iatt[:, :, 1]

    wv1 = Wv_i[1].astype(BF16)
    wihT = w_ih.T.astype(BF16)
    whhT = w_hh.T.astype(BF16)
    bih = b_ih.reshape(1, 3 * NHID)
    bhh = b_hh.reshape(1, 3 * NHID)
    eye = jnp.eye(NB_OUT, dtype=F32)
    def bd_proj(w):
        return (eye[:, None, :, None] * w[:, :, None, :]).reshape(NHID, ATT_OUT).astype(BF16)
    wqkv = jnp.concatenate([bd_proj(Wq_c), bd_proj(Wk_c), bd_proj(Wv_c)], axis=1)
    def bd_fc(w):
        wt = w.T
        return (eye[:, None, :, None] * wt[None, :, None, :]
                ).reshape(ATT_OUT, NHID).astype(BF16)
    fgbd = jnp.concatenate([bd_fc(fc_w), bd_fc(gate_w)], axis=1)
    fgb = jnp.concatenate([jnp.tile(fc_b, (NB_OUT,)),
                           jnp.tile(gate_b, (NB_OUT,))]).reshape(1, 2 * NHID)
    rmat = jnp.asarray(_R_NP, dtype=BF16)
    rtmat = jnp.asarray(_R_NP.T, dtype=BF16)
    e8 = jnp.asarray(_E8_NP, dtype=BF16)

    grid = (bsz // TILE,)
    tspec = lambda w: pl.BlockSpec((TILE, w), lambda i: (i, 0))
    hx_out, cx_out, mask = pl.pallas_call(
        _fused_kernel,
        grid=grid,
        in_specs=[
            tspec(NINP), tspec(NHID), tspec(NHID), tspec(NB_OUT), tspec(NB_OUT),
            _full((NINP, ATT_OUT)), _full((NB_OUT, NHID)),
            _full((ATT_OUT * NB_OUT, 3 * NHID)), _full((NHID, 3 * NHID)),
            _full((1, 3 * NHID)), _full((1, 3 * NHID)),
            _full((NHID, 3 * ATT_OUT)),
            _full((ATT_OUT, 32)), _full((32, ATT_OUT)),
            _full((ATT_OUT, 2 * NHID)),
            _full((1, 2 * NHID)),
        ],
        out_specs=[tspec(NHID), tspec(NHID), tspec(NHID)],
        out_shape=[
            jax.ShapeDtypeStruct((bsz, NHID), F32),
            jax.ShapeDtypeStruct((bsz, NHID), F32),
            jax.ShapeDtypeStruct((bsz, NHID), F32),
        ],
        compiler_params=pltpu.CompilerParams(dimension_semantics=("parallel",)),
    )(inp, hx, cx, score, a1,
      wv1, e8, wihT, whhT, bih, bhh,
      wqkv, rmat, rtmat,
      fgbd, fgb)
    return hx_out, cx_out, mask


# speedup vs baseline: 1.1115x; 1.1115x over previous
import functools

import jax
import jax.numpy as jnp
import numpy as np
from jax.experimental import pallas as pl
from jax.experimental.pallas import tpu as pltpu

B = 4096
NINP = 512
NHID = 1024
NB_OUT = 8
TOPK = 4
BS_OUT = NHID // NB_OUT
ATT_OUT = BS_OUT * 4
NH = 4
DH = 16
TILE = 256
F32 = jnp.float32
BF16 = jnp.bfloat16

_R_NP = (np.arange(512)[:, None] // DH == np.arange(32)[None, :]).astype(np.float32)
_E8_NP = (np.arange(8)[:, None] == np.arange(1024)[None, :] // BS_OUT).astype(np.float32)


def _fused_kernel(x_ref, h_ref, c_ref, sc_ref, a1_ref,
                  wv1_ref, e8_ref, wih_ref, whh_ref, bih_ref, bhh_ref,
                  wqkv_ref, r_ref, rt_ref,
                  fgbd_ref, fgb_ref,
                  hx_out_ref, cx_out_ref, mask_out_ref):
    x = x_ref[:]
    h = h_ref[:]
    c = c_ref[:]
    sc = sc_ref[:]
    a1 = a1_ref[:]

    v1 = jnp.dot(x.astype(BF16), wv1_ref[:], preferred_element_type=F32)
    cols = [(a1[:, b:b + 1] * v1).astype(BF16) for b in range(NB_OUT)]
    inp_flat = jnp.concatenate(cols, axis=1)

    gi = jnp.dot(inp_flat, wih_ref[:], preferred_element_type=F32) + bih_ref[:]
    gh = jnp.dot(h.astype(BF16), whh_ref[:], preferred_element_type=F32) + bhh_ref[:]

    rank = jnp.zeros((TILE, NB_OUT), jnp.int32)
    bidx = jax.lax.broadcasted_iota(jnp.int32, (TILE, NB_OUT), 1)
    for j in range(NB_OUT):
        scj = sc[:, j:j + 1]
        ahead = (scj > sc) | ((scj == sc) & (j < bidx))
        rank = rank + ahead.astype(jnp.int32)
    maskblk = (rank >= (NB_OUT - TOPK)).astype(F32)
    mask = jnp.dot(maskblk.astype(BF16), e8_ref[:],
                   preferred_element_type=F32)
    r = jax.nn.sigmoid(gi[:, :NHID] + gh[:, :NHID])
    z = jax.nn.sigmoid(gi[:, NHID:2 * NHID] + gh[:, NHID:2 * NHID])
    n = jnp.tanh(gi[:, 2 * NHID:] + r * gh[:, 2 * NHID:])
    hgru = (1.0 - z) * n + z * h

    hgb = hgru.astype(BF16)
    qkv = jnp.dot(hgb, wqkv_ref[:], preferred_element_type=F32)
    qc = qkv[:, :ATT_OUT]
    kc = qkv[:, ATT_OUT:2 * ATT_OUT]
    vc = qkv[:, 2 * ATT_OUT:]
    logits = []
    for k in range(NB_OUT):
        kct = jnp.tile(kc[:, k * 64:(k + 1) * 64], (1, NB_OUT))
        prod = qc * kct
        logits.append(jnp.dot(prod.astype(BF16), r_ref[:],
                              preferred_element_type=F32) * 0.25)
    m = logits[0]
    for k in range(1, NB_OUT):
        m = jnp.maximum(m, logits[k])
    es = [jnp.exp(lg - m) for lg in logits]
    s = es[0]
    for k in range(1, NB_OUT):
        s = s + es[k]
    sinv = 1.0 / s
    oc = jnp.zeros((TILE, ATT_OUT), F32)
    for k in range(NB_OUT):
        wk = es[k] * sinv
        wexp = jnp.dot(wk.astype(BF16), rt_ref[:],
                       preferred_element_type=F32)
        vct = jnp.tile(vc[:, k * 64:(k + 1) * 64], (1, NB_OUT))
        oc = oc + wexp * vct

    fg = jnp.dot(oc.astype(BF16), fgbd_ref[:], preferred_element_type=F32) + fgb_ref[:]
    fc_out = fg[:, :NHID]
    gate = jax.nn.sigmoid(fg[:, NHID:])
    h_new = hgru + gate * jnp.tanh(fc_out)

    one_m = 1.0 - mask
    hx_out_ref[:] = mask * h_new + one_m * h
    cx_out_ref[:] = mask * hgru + one_m * c
    mask_out_ref[:] = mask


def _full(shape):
    nd = len(shape)
    return pl.BlockSpec(shape, lambda i: (0,) * nd)


def kernel(inp, hx, cx, step, Wq_i, Wk_i, Wv_i, Wq_c, Wk_c, Wv_c, fc_w, fc_b, gate_w, gate_b, w_ih, w_hh, b_ih, b_hh):
    del step
    bsz = inp.shape[0]
    q = jnp.einsum('bnd,nde->bne', hx.reshape(bsz, NB_OUT, BS_OUT), Wq_i)
    k1 = jnp.einsum('bd,de->be', inp, Wk_i[1])
    l1 = jnp.einsum('bqd,bd->bq', q, k1) / np.sqrt(64.0)
    mx = jnp.maximum(l1, 0.0)
    u0 = jnp.exp(0.0 - mx)
    u1 = jnp.exp(l1 - mx)
    denom = u0 + u1
    score = u0 / denom
    a1 = u1 / denom

    wv1 = Wv_i[1].astype(BF16)
    wihT = w_ih.T.astype(BF16)
    whhT = w_hh.T.astype(BF16)
    bih = b_ih.reshape(1, 3 * NHID)
    bhh = b_hh.reshape(1, 3 * NHID)
    eye = jnp.eye(NB_OUT, dtype=F32)
    def bd_proj(w):
        return (eye[:, None, :, None] * w[:, :, None, :]).reshape(NHID, ATT_OUT).astype(BF16)
    wqkv = jnp.concatenate([bd_proj(Wq_c), bd_proj(Wk_c), bd_proj(Wv_c)], axis=1)
    def bd_fc(w):
        wt = w.T
        return (eye[:, None, :, None] * wt[None, :, None, :]
                ).reshape(ATT_OUT, NHID).astype(BF16)
    fgbd = jnp.concatenate([bd_fc(fc_w), bd_fc(gate_w)], axis=1)
    fgb = jnp.concatenate([jnp.tile(fc_b, (NB_OUT,)),
                           jnp.tile(gate_b, (NB_OUT,))]).reshape(1, 2 * NHID)
    rmat = jnp.asarray(_R_NP, dtype=BF16)
    rtmat = jnp.asarray(_R_NP.T, dtype=BF16)
    e8 = jnp.asarray(_E8_NP, dtype=BF16)

    grid = (bsz // TILE,)
    tspec = lambda w: pl.BlockSpec((TILE, w), lambda i: (i, 0))
    hx_out, cx_out, mask = pl.pallas_call(
        _fused_kernel,
        grid=grid,
        in_specs=[
            tspec(NINP), tspec(NHID), tspec(NHID), tspec(NB_OUT), tspec(NB_OUT),
            _full((NINP, ATT_OUT)), _full((NB_OUT, NHID)),
            _full((ATT_OUT * NB_OUT, 3 * NHID)), _full((NHID, 3 * NHID)),
            _full((1, 3 * NHID)), _full((1, 3 * NHID)),
            _full((NHID, 3 * ATT_OUT)),
            _full((ATT_OUT, 32)), _full((32, ATT_OUT)),
            _full((ATT_OUT, 2 * NHID)),
            _full((1, 2 * NHID)),
        ],
        out_specs=[tspec(NHID), tspec(NHID), tspec(NHID)],
        out_shape=[
            jax.ShapeDtypeStruct((bsz, NHID), F32),
            jax.ShapeDtypeStruct((bsz, NHID), F32),
            jax.ShapeDtypeStruct((bsz, NHID), F32),
        ],
        compiler_params=pltpu.CompilerParams(dimension_semantics=("parallel",)),
    )(inp, hx, cx, score, a1,
      wv1, e8, wihT, whhT, bih, bhh,
      wqkv, rmat, rtmat,
      fgbd, fgb)
    return hx_out, cx_out, mask
